# Initial kernel scaffold; baseline (speedup 1.0000x reference)
import jax
import jax.numpy as jnp
from jax.experimental import pallas as pl


def kernel(h, c, edge_index, node_atts, edges, params):
    raise NotImplementedError("write your pallas kernel here")


# speedup vs baseline: 3.8232x; 3.8232x over previous
import functools

import jax
import jax.numpy as jnp
from jax import lax
from jax.experimental import pallas as pl
from jax.experimental.pallas import tpu as pltpu
from jax.experimental.pallas import tpu_sc as plsc

NDIM = 128
GDIM = 128
HID = 64
B = 100
IDX = 100
N = B * IDX
E = 320000
NUM_ATTS = 8
ALPHA = 0.5

NC = 2
NS = 16
K = 128
GC = 8
NCH = 160
NGRP = NCH // GC
ETP = NCH * K
EP = ETP * NS
RPT = 632
NPAD = RPT * NS
DUMP = N
DEGW = 16

F32 = jnp.float32


def _mmT(x, w):
    return lax.dot_general(x, w, (((1,), (1,)), ((), ())),
                           preferred_element_type=F32)


def _sc_scatter_body(af_hbm, ab_hbm, isrc_hbm, idst_hbm, zrow_hbm, out_hbm,
                     shared, isrc_v, idst_v, rows, gsem):
    cid = lax.axis_index("c")
    sid = lax.axis_index("s")
    pltpu.sync_copy(zrow_hbm, shared.at[pl.ds(sid * RPT, RPT)])
    plsc.subcore_barrier()

    def run(tab):
        def group(g, carry):
            pltpu.sync_copy(isrc_hbm.at[cid, sid, pl.ds(g * GC, GC)], isrc_v)
            pltpu.sync_copy(idst_hbm.at[cid, sid, pl.ds(g * GC, GC)], idst_v)

            def chunk(j, c2):
                pltpu.async_copy(tab.at[isrc_v.at[j]], rows, gsem).wait()
                pltpu.sync_copy(rows, shared.at[idst_v.at[j]], add=True)
                return c2
            return lax.fori_loop(0, GC, chunk, carry)
        lax.fori_loop(0, NGRP, group, 0)

    @pl.when(cid == 0)
    def _():
        run(af_hbm)

    @pl.when(cid == 1)
    def _():
        run(ab_hbm)

    plsc.subcore_barrier()
    pltpu.sync_copy(shared.at[pl.ds(sid * RPT, RPT)],
                    out_hbm.at[cid, pl.ds(sid * RPT, RPT)])


def _sc_scatter(a_f, a_b, isrc, idst, zrow):
    fn = pl.kernel(
        _sc_scatter_body,
        out_type=jax.ShapeDtypeStruct((NC, NPAD, NDIM), F32),
        mesh=plsc.VectorSubcoreMesh(core_axis_name="c", subcore_axis_name="s"),
        scratch_types=[
            pltpu.VMEM_SHARED((NPAD, NDIM), F32),
            pltpu.VMEM((GC, K), jnp.int32),
            pltpu.VMEM((GC, K), jnp.int32),
            pltpu.VMEM((K, NDIM), F32),
            pltpu.SemaphoreType.DMA,
        ],
    )
    return fn(a_f, a_b, isrc, idst, zrow)


def _sc_deg_body(idst_hbm, ones_hbm, zrow_hbm, out_hbm,
                 shared, idst_v, ones_v):
    cid = lax.axis_index("c")
    sid = lax.axis_index("s")
    pltpu.sync_copy(idst_hbm.at[cid, sid], idst_v)
    pltpu.sync_copy(ones_hbm, ones_v)
    pltpu.sync_copy(zrow_hbm, shared.at[pl.ds(sid * RPT, RPT)])
    plsc.subcore_barrier()

    def chunk(j, carry):
        pltpu.sync_copy(ones_v, shared.at[idst_v.at[j]], add=True)
        return carry
    lax.fori_loop(0, NCH, chunk, 0)

    plsc.subcore_barrier()
    pltpu.sync_copy(shared.at[pl.ds(sid * RPT, RPT)],
                    out_hbm.at[cid, pl.ds(sid * RPT, RPT)])


def _sc_deg(idst, ones, zrow):
    fn = pl.kernel(
        _sc_deg_body,
        out_type=jax.ShapeDtypeStruct((NC, NPAD, NDIM), F32),
        mesh=plsc.VectorSubcoreMesh(core_axis_name="c", subcore_axis_name="s"),
        scratch_types=[
            pltpu.VMEM_SHARED((NPAD, NDIM), F32),
            pltpu.VMEM((NCH, K), jnp.int32),
            pltpu.VMEM((K, NDIM), F32),
        ],
    )
    return fn(idst, ones, zrow)


R = 1000
NBLK = N // R


def _full(shape):
    nd = len(shape)
    return pl.BlockSpec(shape, lambda *a, _nd=nd: (0,) * _nd)


def _pre_body(hp_ref, wsf, wdf, mbf, wsb, wdb, mbb, af, cf, ab, cb):
    hp = hp_ref[...]
    hf = hp[:, :HID]
    hb = hp[:, HID:]
    af[...] = _mmT(hf, wsf[...])
    cf[...] = _mmT(hf, wdf[...]) + mbf[...]
    ab[...] = _mmT(hb, wsb[...])
    cb[...] = _mmT(hb, wdb[...]) + mbb[...]


def _tc_pre(h_flat, wsf, wdf, mbf, wsb, wdb, mbb):
    row = pl.BlockSpec((R, NDIM), lambda i: (i, 0))
    o = jax.ShapeDtypeStruct((N, NDIM), F32)
    return pl.pallas_call(
        _pre_body,
        grid=(NBLK,),
        in_specs=[row, _full((NDIM, HID)), _full((NDIM, HID)), _full((1, NDIM)),
                  _full((NDIM, HID)), _full((NDIM, HID)), _full((1, NDIM))],
        out_specs=[row, row, row, row],
        out_shape=[o, o, o, o],
    )(h_flat, wsf, wdf, mbf, wsb, wdb, mbb)


def _gru_half(hh, x, wih, whh, bih, bhh):
    r = jax.nn.sigmoid(_mmT(x, wih[0]) + bih[0][None, :]
                       + _mmT(hh, whh[0]) + bhh[0][None, :])
    z = jax.nn.sigmoid(_mmT(x, wih[1]) + bih[1][None, :]
                       + _mmT(hh, whh[1]) + bhh[1][None, :])
    n = jnp.tanh(_mmT(x, wih[2]) + bih[2][None, :]
                 + r * (_mmT(hh, whh[2]) + bhh[2][None, :]))
    return (1.0 - z) * n + z * hh


def _layer_body(with_next, hp_ref, scat_ref, deg_ref, cf_ref, cb_ref,
                wihf, whhf, bihf, bhhf, wihb, whhb, bihb, bhhb,
                *rest):
    hp = hp_ref[...]
    scat = scat_ref[...]
    deg = deg_ref[...]
    hf = hp[:, :HID]
    hb = hp[:, HID:]
    xf = scat[0] + deg[0] * cf_ref[...]
    xb = scat[1] + deg[1] * cb_ref[...]
    hf2 = _gru_half(hf, xf, wihf[...], whhf[...], bihf[...], bhhf[...])
    hb2 = _gru_half(hb, xb, wihb[...], whhb[...], bihb[...], bhhb[...])
    if with_next:
        (wsf, wdf, mbf, wsb, wdb, mbb,
         hp_out, af, cf, ab, cb) = rest
        af[...] = _mmT(hf2, wsf[...])
        cf[...] = _mmT(hf2, wdf[...]) + mbf[...]
        ab[...] = _mmT(hb2, wsb[...])
        cb[...] = _mmT(hb2, wdb[...]) + mbb[...]
    else:
        (hp_out,) = rest
    hp_out[...] = jnp.concatenate([hf2, hb2], axis=1)


def _tc_layer(hp, scat, deg, c_f, c_b, gw, nxt):
    row = pl.BlockSpec((R, NDIM), lambda i: (i, 0))
    scat_s = pl.BlockSpec((NC, R, NDIM), lambda i: (0, i, 0))
    deg_s = pl.BlockSpec((NC, R, 1), lambda i: (0, i, 0))
    wih_s = _full((3, HID, NDIM))
    whh_s = _full((3, HID, HID))
    b_s = _full((3, HID))
    in_specs = [row, scat_s, deg_s, row, row,
                wih_s, whh_s, b_s, b_s, wih_s, whh_s, b_s, b_s]
    args = [hp, scat, deg, c_f, c_b] + list(gw)
    o = jax.ShapeDtypeStruct((N, NDIM), F32)
    if nxt is not None:
        in_specs += [_full((NDIM, HID)), _full((NDIM, HID)), _full((1, NDIM)),
                     _full((NDIM, HID)), _full((NDIM, HID)), _full((1, NDIM))]
        args += list(nxt)
        out_specs = [row, row, row, row, row]
        out_shape = [o, o, o, o, o]
    else:
        out_specs = [row]
        out_shape = [o]
    return pl.pallas_call(
        functools.partial(_layer_body, nxt is not None),
        grid=(NBLK,),
        in_specs=in_specs,
        out_specs=out_specs,
        out_shape=out_shape,
    )(*args)


def _head1_body(hn_ref, fm_ref, fmb_ref, gm_ref, gmb_ref, out_ref):
    hn = hn_ref[...]
    u = _mmT(hn, fm_ref[0]) + fmb_ref[0]
    g = jax.nn.sigmoid(jnp.sum(hn * gm_ref[0], axis=1, keepdims=True)
                       + gmb_ref[0, 0, 0])
    gated = (u * g).reshape(R // IDX, IDX, GDIM)
    out_ref[0, 0] = jnp.sum(gated, axis=1)


def _tc_head1(hn, fm_s, fmb_s, gm_s, gmb_s):
    return pl.pallas_call(
        _head1_body,
        grid=(2, NBLK),
        in_specs=[pl.BlockSpec((R, NDIM), lambda g, i: (i, 0)),
                  pl.BlockSpec((1, GDIM, NDIM), lambda g, i: (g, 0, 0)),
                  pl.BlockSpec((1, 1, GDIM), lambda g, i: (g, 0, 0)),
                  pl.BlockSpec((1, 1, NDIM), lambda g, i: (g, 0, 0)),
                  pl.BlockSpec((1, 1, 1), lambda g, i: (g, 0, 0))],
        out_specs=pl.BlockSpec((1, 1, R // IDX, GDIM),
                               lambda g, i: (g, i, 0, 0)),
        out_shape=jax.ShapeDtypeStruct((2, NBLK, R // IDX, GDIM), F32),
    )(hn, fm_s, fmb_s, gm_s, gmb_s)


def _head2_body(hg_ref, c_ref, na_ref,
                fw1, fw2, fanb, f2w, f2b, ninits,
                fiw1, fiw2, fiw3, fib, fi2w, fi2b,
                w2, w3, w4, fs1b,
                hv_out, nl_out, t2_out, t34_out):
    hg = hg_ref[0]
    hgi = hg_ref[1]
    c = c_ref[...]
    s = _mmT(hg, fw1[...]) + _mmT(c, fw2[...]) + fanb[...]
    ns = _mmT(jax.nn.relu(s), f2w[...]) + f2b[...]
    m = jnp.max(ns, axis=1, keepdims=True)
    lse = m + jnp.log(jnp.sum(jnp.exp(ns - m), axis=1, keepdims=True))
    logp = ns - lse
    iota = lax.broadcasted_iota(jnp.int32, (B, NUM_ATTS), 1)
    oh = (iota == na_ref[...]).astype(F32)
    nl_out[...] = -jnp.sum(logp * oh, axis=1, keepdims=True)
    e = jnp.dot(oh, ninits[...], preferred_element_type=F32)
    pre = (_mmT(e, fiw1[...]) + _mmT(hgi, fiw2[...]) + _mmT(c, fiw3[...])
           + fib[...])
    hv = _mmT(jax.nn.relu(pre), fi2w[...]) + fi2b[...]
    hv_out[...] = hv
    t2_out[...] = _mmT(hv, w2[...])
    t34_out[...] = _mmT(hg, w3[...]) + _mmT(c, w4[...]) + fs1b[...]


def _tc_head2(hg_s, c, na, weights):
    D2 = NDIM + GDIM
    shapes = [(2, B, GDIM), (B, GDIM), (B, 1),
              (GDIM, GDIM), (GDIM, GDIM), (1, GDIM),
              (NUM_ATTS, GDIM), (1, NUM_ATTS), (NUM_ATTS, NDIM),
              (D2, NDIM), (D2, GDIM), (D2, GDIM), (1, D2),
              (NDIM, D2), (1, NDIM),
              (D2, GDIM), (D2, GDIM), (D2, GDIM), (1, D2)]
    return pl.pallas_call(
        _head2_body,
        in_specs=[_full(s) for s in shapes],
        out_specs=[_full((B, NDIM)), _full((B, 1)),
                   _full((B, D2)), _full((B, D2))],
        out_shape=[jax.ShapeDtypeStruct((B, NDIM), F32),
                   jax.ShapeDtypeStruct((B, 1), F32),
                   jax.ShapeDtypeStruct((B, D2), F32),
                   jax.ShapeDtypeStruct((B, D2), F32)],
    )(hg_s, c, na, *weights)


def _head3_body(hn_ref, w1_ref, t2_ref, t34_ref, f2w_ref, f2b_ref,
                edges_ref, nl_ref, loss_out):
    G = R // IDX
    t1 = _mmT(hn_ref[...], w1_ref[...])
    s2 = (t1.reshape(G, IDX, NDIM + GDIM) + t2_ref[0][:, None, :]
          + t34_ref[...][None, :, :])
    rs = jax.nn.relu(s2)
    es = jnp.sum(rs * f2w_ref[...][None, :, :], axis=2) + f2b_ref[0, 0]
    bce = (jnp.maximum(es, 0.0) - es * edges_ref[0]
           + jnp.log(1.0 + jnp.exp(-jnp.abs(es))))
    el = jnp.mean(bce, axis=1, keepdims=True)
    loss_out[0] = 2.0 * ((1.0 - ALPHA) * nl_ref[0] + ALPHA * el)


def _tc_head3(hn, w1, t2, t34, f2w, f2b, edges, nl):
    D2 = NDIM + GDIM
    G = R // IDX
    return pl.pallas_call(
        _head3_body,
        grid=(NBLK,),
        in_specs=[pl.BlockSpec((R, NDIM), lambda i: (i, 0)),
                  _full((D2, NDIM)),
                  pl.BlockSpec((1, G, D2), lambda i: (i, 0, 0)),
                  _full((B, D2)),
                  _full((1, D2)),
                  _full((1, 1)),
                  pl.BlockSpec((1, G, IDX), lambda i: (i, 0, 0)),
                  pl.BlockSpec((1, G, 1), lambda i: (i, 0, 0))],
        out_specs=pl.BlockSpec((1, G, 1), lambda i: (i, 0, 0)),
        out_shape=jax.ShapeDtypeStruct((NBLK, G, 1), F32),
    )(hn, w1, t2.reshape(NBLK, G, D2), t34, f2w, f2b,
      edges.reshape(NBLK, G, IDX), nl.reshape(NBLK, G, 1))


def _gru_weights(p):
    return (p['Wih'].reshape(3, HID, NDIM), p['Whh'].reshape(3, HID, HID),
            p['bih'].reshape(3, HID), p['bhh'].reshape(3, HID))


def _msg_weights(p):
    return (p['msg_W'][:, :HID], p['msg_W'][:, HID:], p['msg_b'][None, :])


def kernel(h, c, edge_index, node_atts, edges, params):
    h_flat = h.reshape(-1, NDIM)
    ei = edge_index.astype(jnp.int32)
    s0, d0 = ei[0], ei[1]

    padz = jnp.zeros((EP - E,), jnp.int32)
    padd = jnp.full((EP - E,), DUMP, jnp.int32)
    isrc = jnp.stack([jnp.concatenate([s0, padz]),
                      jnp.concatenate([d0, padz])]).reshape(NC, NS, NCH, K)
    idst = jnp.stack([jnp.concatenate([d0, padd]),
                      jnp.concatenate([s0, padd])]).reshape(NC, NS, NCH, K)

    zrow = jnp.zeros((RPT, NDIM), F32)
    onesw = jnp.ones((K, NDIM), F32)

    deg = _sc_deg(idst, onesw, zrow)[:, :, :1]

    pf0, pb0 = params['fwd_layers'][0], params['bwd_layers'][0]
    pf1, pb1 = params['fwd_layers'][1], params['bwd_layers'][1]

    wsf0, wdf0, mbf0 = _msg_weights(pf0)
    wsb0, wdb0, mbb0 = _msg_weights(pb0)
    a_f0, c_f0, a_b0, c_b0 = _tc_pre(h_flat, wsf0, wdf0, mbf0,
                                     wsb0, wdb0, mbb0)

    scat0 = _sc_scatter(a_f0, a_b0, isrc, idst, zrow)

    gw0 = _gru_weights(pf0) + _gru_weights(pb0)
    nxt1 = _msg_weights(pf1) + _msg_weights(pb1)
    h1, a_f1, c_f1, a_b1, c_b1 = _tc_layer(h_flat, scat0, deg,
                                           c_f0, c_b0, gw0, nxt1)

    scat1 = _sc_scatter(a_f1, a_b1, isrc, idst, zrow)

    gw1 = _gru_weights(pf1) + _gru_weights(pb1)
    (hn,) = _tc_layer(h1, scat1, deg, c_f1, c_b1, gw1, None)

    pg, pgi = params['graph_emb'], params['graph_emb_init']
    fm_s = jnp.stack([pg['fm_W'], pgi['fm_W']])
    fmb_s = jnp.stack([pg['fm_b'], pgi['fm_b']])[:, None, :]
    gm_s = jnp.stack([pg['gm_W'][0], pgi['gm_W'][0]])[:, None, :]
    gmb_s = jnp.stack([pg['gm_b'], pgi['gm_b']])[:, :, None]
    hg_s = _tc_head1(hn, fm_s, fmb_s, gm_s, gmb_s).reshape(2, B, GDIM)

    D2 = NDIM + GDIM
    f1 = params['fs1_W']
    h2w = (params['fan_W'][:, :GDIM], params['fan_W'][:, GDIM:],
           params['fan_b'][None, :],
           params['fan2_W'], params['fan2_b'][None, :],
           params['node_inits'],
           params['finit_W'][:, :NDIM], params['finit_W'][:, NDIM:NDIM + GDIM],
           params['finit_W'][:, NDIM + GDIM:], params['finit_b'][None, :],
           params['finit2_W'], params['finit2_b'][None, :],
           f1[:, NDIM:NDIM + GDIM], f1[:, NDIM + GDIM:NDIM + 2 * GDIM],
           f1[:, NDIM + 2 * GDIM:], params['fs1_b'][None, :])
    na = node_atts.astype(jnp.int32)[:, None]
    h_v, nl, t2, t34 = _tc_head2(hg_s, c, na, h2w)

    lossb = _tc_head3(hn, f1[:, :NDIM], t2, t34,
                      params['fs2_W'], params['fs2_b'][None, :], edges, nl)

    h_out = jnp.concatenate([hn.reshape(B, IDX, NDIM), h_v[:, None, :]],
                            axis=1)
    return (h_out, lossb.reshape(B))


# speedup vs baseline: 4.5746x; 1.1965x over previous
import functools

import jax
import jax.numpy as jnp
from jax import lax
from jax.experimental import pallas as pl
from jax.experimental.pallas import tpu as pltpu
from jax.experimental.pallas import tpu_sc as plsc

NDIM = 128
GDIM = 128
HID = 64
B = 100
IDX = 100
N = B * IDX
E = 320000
NUM_ATTS = 8
ALPHA = 0.5

NC = 2
NS = 16
K = 128
GC = 8
NCH = 160
NGRP = NCH // GC
ETP = NCH * K
EP = ETP * NS
RPT = 632
NPAD = RPT * NS
DUMP = N
DEGW = 16

F32 = jnp.float32


def _mmT(x, w):
    return lax.dot_general(x, w, (((1,), (1,)), ((), ())),
                           preferred_element_type=F32)


NPAIR = NGRP // 2


def _sc_scatter_body(af_hbm, ab_hbm, isrc_hbm, idst_hbm, zrow_hbm, out_hbm,
                     shared, isrc_a, idst_a, isrc_b, idst_b,
                     rows0, rows1, sem0, sem1):
    cid = lax.axis_index("c")
    sid = lax.axis_index("s")
    pltpu.sync_copy(zrow_hbm, shared.at[pl.ds(sid * RPT, RPT)])
    plsc.subcore_barrier()

    def run(tab):
        rows = (rows0, rows1)
        sems = (sem0, sem1)

        def stage(g, sbuf, dbuf):
            pltpu.sync_copy(isrc_hbm.at[cid, sid, pl.ds(g * GC, GC)], sbuf)
            pltpu.sync_copy(idst_hbm.at[cid, sid, pl.ds(g * GC, GC)], dbuf)

        def issue(sbuf, j, p):
            pltpu.async_copy(tab.at[sbuf.at[j]], rows[p], sems[p])

        def drain(sbuf, j, p):
            pltpu.make_async_copy(tab.at[sbuf.at[j]], rows[p], sems[p]).wait()

        def scat(dbuf, j, p):
            pltpu.sync_copy(rows[p], shared.at[dbuf.at[j]], add=True)

        stage(0, isrc_a, idst_a)
        issue(isrc_a, 0, 0)
        issue(isrc_a, 1, 1)

        def pair(i, carry):
            stage(2 * i + 1, isrc_b, idst_b)
            for j in range(GC):
                p = j % 2
                drain(isrc_a, j, p)
                scat(idst_a, j, p)
                if j + 2 < GC:
                    issue(isrc_a, j + 2, p)
                else:
                    issue(isrc_b, j + 2 - GC, p)

            @pl.when(i < NPAIR - 1)
            def _():
                stage(2 * i + 2, isrc_a, idst_a)

            for j in range(GC):
                p = j % 2
                drain(isrc_b, j, p)
                scat(idst_b, j, p)
                if j + 2 < GC:
                    issue(isrc_b, j + 2, p)
                else:
                    @pl.when(i < NPAIR - 1)
                    def _(j=j, p=p):
                        issue(isrc_a, j + 2 - GC, p)
            return carry
        lax.fori_loop(0, NPAIR, pair, 0)

    @pl.when(cid == 0)
    def _():
        run(af_hbm)

    @pl.when(cid == 1)
    def _():
        run(ab_hbm)

    plsc.subcore_barrier()
    pltpu.sync_copy(shared.at[pl.ds(sid * RPT, RPT)],
                    out_hbm.at[cid, pl.ds(sid * RPT, RPT)])


def _sc_scatter(a_f, a_b, isrc, idst, zrow):
    fn = pl.kernel(
        _sc_scatter_body,
        out_type=jax.ShapeDtypeStruct((NC, NPAD, NDIM), F32),
        mesh=plsc.VectorSubcoreMesh(core_axis_name="c", subcore_axis_name="s"),
        scratch_types=[
            pltpu.VMEM_SHARED((NPAD, NDIM), F32),
            pltpu.VMEM((GC, K), jnp.int32),
            pltpu.VMEM((GC, K), jnp.int32),
            pltpu.VMEM((GC, K), jnp.int32),
            pltpu.VMEM((GC, K), jnp.int32),
            pltpu.VMEM((K, NDIM), F32),
            pltpu.VMEM((K, NDIM), F32),
            pltpu.SemaphoreType.DMA,
            pltpu.SemaphoreType.DMA,
        ],
    )
    return fn(a_f, a_b, isrc, idst, zrow)


def _sc_deg_body(idst_hbm, ones_hbm, zrow_hbm, out_hbm,
                 shared, idst_v, ones_v):
    cid = lax.axis_index("c")
    sid = lax.axis_index("s")
    pltpu.sync_copy(idst_hbm.at[cid, sid], idst_v)
    pltpu.sync_copy(ones_hbm, ones_v)
    pltpu.sync_copy(zrow_hbm, shared.at[pl.ds(sid * RPT, RPT)])
    plsc.subcore_barrier()

    def chunk(j, carry):
        pltpu.sync_copy(ones_v, shared.at[idst_v.at[j]], add=True)
        return carry
    lax.fori_loop(0, NCH, chunk, 0)

    plsc.subcore_barrier()
    pltpu.sync_copy(shared.at[pl.ds(sid * RPT, RPT)],
                    out_hbm.at[cid, pl.ds(sid * RPT, RPT)])


def _sc_deg(idst, ones, zrow):
    fn = pl.kernel(
        _sc_deg_body,
        out_type=jax.ShapeDtypeStruct((NC, NPAD, NDIM), F32),
        mesh=plsc.VectorSubcoreMesh(core_axis_name="c", subcore_axis_name="s"),
        scratch_types=[
            pltpu.VMEM_SHARED((NPAD, NDIM), F32),
            pltpu.VMEM((NCH, K), jnp.int32),
            pltpu.VMEM((K, NDIM), F32),
        ],
    )
    return fn(idst, ones, zrow)


R = 1000
NBLK = N // R


def _full(shape):
    nd = len(shape)
    return pl.BlockSpec(shape, lambda *a, _nd=nd: (0,) * _nd)


def _pre_body(hp_ref, wsf, wdf, mbf, wsb, wdb, mbb, af, cf, ab, cb):
    hp = hp_ref[...]
    hf = hp[:, :HID]
    hb = hp[:, HID:]
    af[...] = _mmT(hf, wsf[...])
    cf[...] = _mmT(hf, wdf[...]) + mbf[...]
    ab[...] = _mmT(hb, wsb[...])
    cb[...] = _mmT(hb, wdb[...]) + mbb[...]


def _tc_pre(h_flat, wsf, wdf, mbf, wsb, wdb, mbb):
    row = pl.BlockSpec((R, NDIM), lambda i: (i, 0))
    o = jax.ShapeDtypeStruct((N, NDIM), F32)
    return pl.pallas_call(
        _pre_body,
        grid=(NBLK,),
        in_specs=[row, _full((NDIM, HID)), _full((NDIM, HID)), _full((1, NDIM)),
                  _full((NDIM, HID)), _full((NDIM, HID)), _full((1, NDIM))],
        out_specs=[row, row, row, row],
        out_shape=[o, o, o, o],
    )(h_flat, wsf, wdf, mbf, wsb, wdb, mbb)


def _gru_half(hh, x, wih, whh, bih, bhh):
    r = jax.nn.sigmoid(_mmT(x, wih[0]) + bih[0][None, :]
                       + _mmT(hh, whh[0]) + bhh[0][None, :])
    z = jax.nn.sigmoid(_mmT(x, wih[1]) + bih[1][None, :]
                       + _mmT(hh, whh[1]) + bhh[1][None, :])
    n = jnp.tanh(_mmT(x, wih[2]) + bih[2][None, :]
                 + r * (_mmT(hh, whh[2]) + bhh[2][None, :]))
    return (1.0 - z) * n + z * hh


def _layer_body(with_next, hp_ref, scat_ref, deg_ref, cf_ref, cb_ref,
                wihf, whhf, bihf, bhhf, wihb, whhb, bihb, bhhb,
                *rest):
    hp = hp_ref[...]
    scat = scat_ref[...]
    deg = deg_ref[...]
    hf = hp[:, :HID]
    hb = hp[:, HID:]
    xf = scat[0] + deg[0] * cf_ref[...]
    xb = scat[1] + deg[1] * cb_ref[...]
    hf2 = _gru_half(hf, xf, wihf[...], whhf[...], bihf[...], bhhf[...])
    hb2 = _gru_half(hb, xb, wihb[...], whhb[...], bihb[...], bhhb[...])
    if with_next:
        (wsf, wdf, mbf, wsb, wdb, mbb,
         hp_out, af, cf, ab, cb) = rest
        af[...] = _mmT(hf2, wsf[...])
        cf[...] = _mmT(hf2, wdf[...]) + mbf[...]
        ab[...] = _mmT(hb2, wsb[...])
        cb[...] = _mmT(hb2, wdb[...]) + mbb[...]
    else:
        (hp_out,) = rest
    hp_out[...] = jnp.concatenate([hf2, hb2], axis=1)


def _tc_layer(hp, scat, deg, c_f, c_b, gw, nxt):
    row = pl.BlockSpec((R, NDIM), lambda i: (i, 0))
    scat_s = pl.BlockSpec((NC, R, NDIM), lambda i: (0, i, 0))
    deg_s = pl.BlockSpec((NC, R, 1), lambda i: (0, i, 0))
    wih_s = _full((3, HID, NDIM))
    whh_s = _full((3, HID, HID))
    b_s = _full((3, HID))
    in_specs = [row, scat_s, deg_s, row, row,
                wih_s, whh_s, b_s, b_s, wih_s, whh_s, b_s, b_s]
    args = [hp, scat, deg, c_f, c_b] + list(gw)
    o = jax.ShapeDtypeStruct((N, NDIM), F32)
    if nxt is not None:
        in_specs += [_full((NDIM, HID)), _full((NDIM, HID)), _full((1, NDIM)),
                     _full((NDIM, HID)), _full((NDIM, HID)), _full((1, NDIM))]
        args += list(nxt)
        out_specs = [row, row, row, row, row]
        out_shape = [o, o, o, o, o]
    else:
        out_specs = [row]
        out_shape = [o]
    return pl.pallas_call(
        functools.partial(_layer_body, nxt is not None),
        grid=(NBLK,),
        in_specs=in_specs,
        out_specs=out_specs,
        out_shape=out_shape,
    )(*args)


def _head1_body(hn_ref, fm_ref, fmb_ref, gm_ref, gmb_ref, out_ref):
    hn = hn_ref[...]
    u = _mmT(hn, fm_ref[0]) + fmb_ref[0]
    g = jax.nn.sigmoid(jnp.sum(hn * gm_ref[0], axis=1, keepdims=True)
                       + gmb_ref[0, 0, 0])
    gated = (u * g).reshape(R // IDX, IDX, GDIM)
    out_ref[0, 0] = jnp.sum(gated, axis=1)


def _tc_head1(hn, fm_s, fmb_s, gm_s, gmb_s):
    return pl.pallas_call(
        _head1_body,
        grid=(2, NBLK),
        in_specs=[pl.BlockSpec((R, NDIM), lambda g, i: (i, 0)),
                  pl.BlockSpec((1, GDIM, NDIM), lambda g, i: (g, 0, 0)),
                  pl.BlockSpec((1, 1, GDIM), lambda g, i: (g, 0, 0)),
                  pl.BlockSpec((1, 1, NDIM), lambda g, i: (g, 0, 0)),
                  pl.BlockSpec((1, 1, 1), lambda g, i: (g, 0, 0))],
        out_specs=pl.BlockSpec((1, 1, R // IDX, GDIM),
                               lambda g, i: (g, i, 0, 0)),
        out_shape=jax.ShapeDtypeStruct((2, NBLK, R // IDX, GDIM), F32),
    )(hn, fm_s, fmb_s, gm_s, gmb_s)


def _head2_body(hg_ref, c_ref, na_ref,
                fw1, fw2, fanb, f2w, f2b, ninits,
                fiw1, fiw2, fiw3, fib, fi2w, fi2b,
                w2, w3, w4, fs1b,
                hv_out, nl_out, t2_out, t34_out):
    hg = hg_ref[0]
    hgi = hg_ref[1]
    c = c_ref[...]
    s = _mmT(hg, fw1[...]) + _mmT(c, fw2[...]) + fanb[...]
    ns = _mmT(jax.nn.relu(s), f2w[...]) + f2b[...]
    m = jnp.max(ns, axis=1, keepdims=True)
    lse = m + jnp.log(jnp.sum(jnp.exp(ns - m), axis=1, keepdims=True))
    logp = ns - lse
    iota = lax.broadcasted_iota(jnp.int32, (B, NUM_ATTS), 1)
    oh = (iota == na_ref[...]).astype(F32)
    nl_out[...] = -jnp.sum(logp * oh, axis=1, keepdims=True)
    e = jnp.dot(oh, ninits[...], preferred_element_type=F32)
    pre = (_mmT(e, fiw1[...]) + _mmT(hgi, fiw2[...]) + _mmT(c, fiw3[...])
           + fib[...])
    hv = _mmT(jax.nn.relu(pre), fi2w[...]) + fi2b[...]
    hv_out[...] = hv
    t2_out[...] = _mmT(hv, w2[...])
    t34_out[...] = _mmT(hg, w3[...]) + _mmT(c, w4[...]) + fs1b[...]


def _tc_head2(hg_s, c, na, weights):
    D2 = NDIM + GDIM
    shapes = [(2, B, GDIM), (B, GDIM), (B, 1),
              (GDIM, GDIM), (GDIM, GDIM), (1, GDIM),
              (NUM_ATTS, GDIM), (1, NUM_ATTS), (NUM_ATTS, NDIM),
              (D2, NDIM), (D2, GDIM), (D2, GDIM), (1, D2),
              (NDIM, D2), (1, NDIM),
              (D2, GDIM), (D2, GDIM), (D2, GDIM), (1, D2)]
    return pl.pallas_call(
        _head2_body,
        in_specs=[_full(s) for s in shapes],
        out_specs=[_full((B, NDIM)), _full((B, 1)),
                   _full((B, D2)), _full((B, D2))],
        out_shape=[jax.ShapeDtypeStruct((B, NDIM), F32),
                   jax.ShapeDtypeStruct((B, 1), F32),
                   jax.ShapeDtypeStruct((B, D2), F32),
                   jax.ShapeDtypeStruct((B, D2), F32)],
    )(hg_s, c, na, *weights)


def _head3_body(hn_ref, w1_ref, t2_ref, t34_ref, f2w_ref, f2b_ref,
                edges_ref, nl_ref, loss_out):
    G = R // IDX
    t1 = _mmT(hn_ref[...], w1_ref[...])
    s2 = (t1.reshape(G, IDX, NDIM + GDIM) + t2_ref[0][:, None, :]
          + t34_ref[...][None, :, :])
    rs = jax.nn.relu(s2)
    es = jnp.sum(rs * f2w_ref[...][None, :, :], axis=2) + f2b_ref[0, 0]
    bce = (jnp.maximum(es, 0.0) - es * edges_ref[0]
           + jnp.log(1.0 + jnp.exp(-jnp.abs(es))))
    el = jnp.mean(bce, axis=1, keepdims=True)
    loss_out[0] = 2.0 * ((1.0 - ALPHA) * nl_ref[0] + ALPHA * el)


def _tc_head3(hn, w1, t2, t34, f2w, f2b, edges, nl):
    D2 = NDIM + GDIM
    G = R // IDX
    return pl.pallas_call(
        _head3_body,
        grid=(NBLK,),
        in_specs=[pl.BlockSpec((R, NDIM), lambda i: (i, 0)),
                  _full((D2, NDIM)),
                  pl.BlockSpec((1, G, D2), lambda i: (i, 0, 0)),
                  _full((B, D2)),
                  _full((1, D2)),
                  _full((1, 1)),
                  pl.BlockSpec((1, G, IDX), lambda i: (i, 0, 0)),
                  pl.BlockSpec((1, G, 1), lambda i: (i, 0, 0))],
        out_specs=pl.BlockSpec((1, G, 1), lambda i: (i, 0, 0)),
        out_shape=jax.ShapeDtypeStruct((NBLK, G, 1), F32),
    )(hn, w1, t2.reshape(NBLK, G, D2), t34, f2w, f2b,
      edges.reshape(NBLK, G, IDX), nl.reshape(NBLK, G, 1))


def _gru_weights(p):
    return (p['Wih'].reshape(3, HID, NDIM), p['Whh'].reshape(3, HID, HID),
            p['bih'].reshape(3, HID), p['bhh'].reshape(3, HID))


def _msg_weights(p):
    return (p['msg_W'][:, :HID], p['msg_W'][:, HID:], p['msg_b'][None, :])


def kernel(h, c, edge_index, node_atts, edges, params):
    h_flat = h.reshape(-1, NDIM)
    ei = edge_index.astype(jnp.int32)
    s0, d0 = ei[0], ei[1]

    padz = jnp.zeros((EP - E,), jnp.int32)
    padd = jnp.full((EP - E,), DUMP, jnp.int32)
    isrc = jnp.stack([jnp.concatenate([s0, padz]),
                      jnp.concatenate([d0, padz])]).reshape(NC, NS, NCH, K)
    idst = jnp.stack([jnp.concatenate([d0, padd]),
                      jnp.concatenate([s0, padd])]).reshape(NC, NS, NCH, K)

    zrow = jnp.zeros((RPT, NDIM), F32)
    onesw = jnp.ones((K, NDIM), F32)

    deg = _sc_deg(idst, onesw, zrow)[:, :, :1]

    pf0, pb0 = params['fwd_layers'][0], params['bwd_layers'][0]
    pf1, pb1 = params['fwd_layers'][1], params['bwd_layers'][1]

    wsf0, wdf0, mbf0 = _msg_weights(pf0)
    wsb0, wdb0, mbb0 = _msg_weights(pb0)
    a_f0, c_f0, a_b0, c_b0 = _tc_pre(h_flat, wsf0, wdf0, mbf0,
                                     wsb0, wdb0, mbb0)

    scat0 = _sc_scatter(a_f0, a_b0, isrc, idst, zrow)

    gw0 = _gru_weights(pf0) + _gru_weights(pb0)
    nxt1 = _msg_weights(pf1) + _msg_weights(pb1)
    h1, a_f1, c_f1, a_b1, c_b1 = _tc_layer(h_flat, scat0, deg,
                                           c_f0, c_b0, gw0, nxt1)

    scat1 = _sc_scatter(a_f1, a_b1, isrc, idst, zrow)

    gw1 = _gru_weights(pf1) + _gru_weights(pb1)
    (hn,) = _tc_layer(h1, scat1, deg, c_f1, c_b1, gw1, None)

    pg, pgi = params['graph_emb'], params['graph_emb_init']
    fm_s = jnp.stack([pg['fm_W'], pgi['fm_W']])
    fmb_s = jnp.stack([pg['fm_b'], pgi['fm_b']])[:, None, :]
    gm_s = jnp.stack([pg['gm_W'][0], pgi['gm_W'][0]])[:, None, :]
    gmb_s = jnp.stack([pg['gm_b'], pgi['gm_b']])[:, :, None]
    hg_s = _tc_head1(hn, fm_s, fmb_s, gm_s, gmb_s).reshape(2, B, GDIM)

    D2 = NDIM + GDIM
    f1 = params['fs1_W']
    h2w = (params['fan_W'][:, :GDIM], params['fan_W'][:, GDIM:],
           params['fan_b'][None, :],
           params['fan2_W'], params['fan2_b'][None, :],
           params['node_inits'],
           params['finit_W'][:, :NDIM], params['finit_W'][:, NDIM:NDIM + GDIM],
           params['finit_W'][:, NDIM + GDIM:], params['finit_b'][None, :],
           params['finit2_W'], params['finit2_b'][None, :],
           f1[:, NDIM:NDIM + GDIM], f1[:, NDIM + GDIM:NDIM + 2 * GDIM],
           f1[:, NDIM + 2 * GDIM:], params['fs1_b'][None, :])
    na = node_atts.astype(jnp.int32)[:, None]
    h_v, nl, t2, t34 = _tc_head2(hg_s, c, na, h2w)

    lossb = _tc_head3(hn, f1[:, :NDIM], t2, t34,
                      params['fs2_W'], params['fs2_b'][None, :], edges, nl)

    h_out = jnp.concatenate([hn.reshape(B, IDX, NDIM), h_v[:, None, :]],
                            axis=1)
    return (h_out, lossb.reshape(B))


# speedup vs baseline: 7.8398x; 1.7138x over previous
import functools

import jax
import jax.numpy as jnp
from jax import lax
from jax.experimental import pallas as pl
from jax.experimental.pallas import tpu as pltpu
from jax.experimental.pallas import tpu_sc as plsc

NDIM = 128
GDIM = 128
HID = 64
B = 100
IDX = 100
N = B * IDX
E = 320000
NUM_ATTS = 8
ALPHA = 0.5

NC = 2
NS = 16
K = 128
GC = 8
NCH = 160
NGRP = NCH // GC
ETP = NCH * K
EP = ETP * NS
RPT = 632
NPAD = RPT * NS
DUMP = N
DEGW = 16

F32 = jnp.float32


def _mmT(x, w):
    return lax.dot_general(x, w, (((1,), (1,)), ((), ())),
                           preferred_element_type=F32)


def _mmT_hi(x, w):
    return lax.dot_general(x, w, (((1,), (1,)), ((), ())),
                           precision=lax.Precision.HIGHEST,
                           preferred_element_type=F32)


def _rd(x):
    return x.astype(jnp.bfloat16).astype(F32)


NPAIR = NGRP // 2
TW = HID
DEPTH = 4


def _sc_scatter_body(af_hbm, ab_hbm, isrc_hbm, idst_hbm, zrow_hbm, out_hbm,
                     shared, isrc_a, idst_a, isrc_b, idst_b,
                     rows0, rows1, rows2, rows3, sem0, sem1, sem2, sem3):
    cid = lax.axis_index("c")
    sid = lax.axis_index("s")
    pltpu.sync_copy(zrow_hbm, shared.at[pl.ds(sid * RPT, RPT)])
    plsc.subcore_barrier()

    def run(tab):
        rows = (rows0, rows1, rows2, rows3)
        sems = (sem0, sem1, sem2, sem3)

        def stage(g, sbuf, dbuf):
            pltpu.sync_copy(isrc_hbm.at[cid, sid, pl.ds(g * GC, GC)], sbuf)
            pltpu.sync_copy(idst_hbm.at[cid, sid, pl.ds(g * GC, GC)], dbuf)

        def issue(sbuf, j, p):
            pltpu.async_copy(tab.at[sbuf.at[j]], rows[p], sems[p])

        def drain(sbuf, j, p):
            pltpu.make_async_copy(tab.at[sbuf.at[j]], rows[p], sems[p]).wait()

        def scat(dbuf, j, p):
            pltpu.sync_copy(rows[p], shared.at[dbuf.at[j]], add=True)

        stage(0, isrc_a, idst_a)
        for p in range(DEPTH):
            issue(isrc_a, p, p)

        def pair(i, carry):
            stage(2 * i + 1, isrc_b, idst_b)
            for j in range(GC):
                p = j % DEPTH
                drain(isrc_a, j, p)
                scat(idst_a, j, p)
                if j + DEPTH < GC:
                    issue(isrc_a, j + DEPTH, p)
                else:
                    issue(isrc_b, j + DEPTH - GC, p)

            @pl.when(i < NPAIR - 1)
            def _():
                stage(2 * i + 2, isrc_a, idst_a)

            for j in range(GC):
                p = j % DEPTH
                drain(isrc_b, j, p)
                scat(idst_b, j, p)
                if j + DEPTH < GC:
                    issue(isrc_b, j + DEPTH, p)
                else:
                    @pl.when(i < NPAIR - 1)
                    def _(j=j, p=p):
                        issue(isrc_a, j + DEPTH - GC, p)
            return carry
        lax.fori_loop(0, NPAIR, pair, 0)

    @pl.when(cid == 0)
    def _():
        run(af_hbm)

    @pl.when(cid == 1)
    def _():
        run(ab_hbm)

    plsc.subcore_barrier()
    pltpu.sync_copy(shared.at[pl.ds(sid * RPT, RPT)],
                    out_hbm.at[cid, pl.ds(sid * RPT, RPT)])


def _sc_scatter(t_f, t_b, isrc, idst, zrow):
    fn = pl.kernel(
        _sc_scatter_body,
        out_type=jax.ShapeDtypeStruct((NC, NPAD, TW), F32),
        compiler_params=pltpu.CompilerParams(use_tc_tiling_on_sc=False),
        mesh=plsc.VectorSubcoreMesh(core_axis_name="c", subcore_axis_name="s"),
        scratch_types=[
            pltpu.VMEM_SHARED((NPAD, TW), F32),
            pltpu.VMEM((GC, K), jnp.int32),
            pltpu.VMEM((GC, K), jnp.int32),
            pltpu.VMEM((GC, K), jnp.int32),
            pltpu.VMEM((GC, K), jnp.int32),
            pltpu.VMEM((K, TW), F32),
            pltpu.VMEM((K, TW), F32),
            pltpu.VMEM((K, TW), F32),
            pltpu.VMEM((K, TW), F32),
            pltpu.SemaphoreType.DMA,
            pltpu.SemaphoreType.DMA,
            pltpu.SemaphoreType.DMA,
            pltpu.SemaphoreType.DMA,
        ],
    )
    return fn(t_f, t_b, isrc, idst, zrow)


def _sc_deg_body(idst_hbm, ones_hbm, zrow_hbm, out_hbm,
                 shared, idst_v, ones_v):
    cid = lax.axis_index("c")
    sid = lax.axis_index("s")
    pltpu.sync_copy(idst_hbm.at[cid, sid], idst_v)
    pltpu.sync_copy(ones_hbm, ones_v)
    pltpu.sync_copy(zrow_hbm, shared.at[pl.ds(sid * RPT, RPT)])
    plsc.subcore_barrier()

    def chunk(j, carry):
        pltpu.sync_copy(ones_v, shared.at[idst_v.at[j]], add=True)
        return carry
    lax.fori_loop(0, NCH, chunk, 0)

    plsc.subcore_barrier()
    pltpu.sync_copy(shared.at[pl.ds(sid * RPT, RPT)],
                    out_hbm.at[cid, pl.ds(sid * RPT, RPT)])


def _sc_deg(idst, ones, zrow):
    fn = pl.kernel(
        _sc_deg_body,
        out_type=jax.ShapeDtypeStruct((NC, NPAD, TW), F32),
        compiler_params=pltpu.CompilerParams(use_tc_tiling_on_sc=False),
        mesh=plsc.VectorSubcoreMesh(core_axis_name="c", subcore_axis_name="s"),
        scratch_types=[
            pltpu.VMEM_SHARED((NPAD, TW), F32),
            pltpu.VMEM((NCH, K), jnp.int32),
            pltpu.VMEM((K, TW), F32),
        ],
    )
    return fn(idst, ones, zrow)


R = 1000
NBLK = N // R


def _full(shape):
    nd = len(shape)
    return pl.BlockSpec(shape, lambda *a, _nd=nd: (0,) * _nd)


def _pre_body(hp_ref, wdf, mbf, wdb, mbb, tf, tb, tgf, tgb, cf, cb):
    hp = hp_ref[...]
    hf = hp[:, :HID]
    hb = hp[:, HID:]
    tf[...] = hf
    tb[...] = hb
    tgf[...] = _rd(hf)
    tgb[...] = _rd(hb)
    cf[...] = _mmT(hf, wdf[...]) + mbf[...]
    cb[...] = _mmT(hb, wdb[...]) + mbb[...]


def _tc_pre(h_flat, wdf, mbf, wdb, mbb):
    row = pl.BlockSpec((R, NDIM), lambda i: (i, 0))
    row64 = pl.BlockSpec((R, HID), lambda i: (i, 0))
    o = jax.ShapeDtypeStruct((N, NDIM), F32)
    o64 = jax.ShapeDtypeStruct((N, HID), F32)
    return pl.pallas_call(
        _pre_body,
        grid=(NBLK,),
        in_specs=[row, _full((NDIM, HID)), _full((1, NDIM)),
                  _full((NDIM, HID)), _full((1, NDIM))],
        out_specs=[row64, row64, row64, row64, row, row],
        out_shape=[o64, o64, o64, o64, o, o],
    )(h_flat, wdf, mbf, wdb, mbb)


def _gru_half(hh, x, wih, whh, bih, bhh):
    r = jax.nn.sigmoid(_mmT(x, wih[0]) + bih[0][None, :]
                       + _mmT(hh, whh[0]) + bhh[0][None, :])
    z = jax.nn.sigmoid(_mmT(x, wih[1]) + bih[1][None, :]
                       + _mmT(hh, whh[1]) + bhh[1][None, :])
    n = jnp.tanh(_mmT(x, wih[2]) + bih[2][None, :]
                 + r * (_mmT(hh, whh[2]) + bhh[2][None, :]))
    return (1.0 - z) * n + z * hh


def _layer_body(with_next, tf_ref, tb_ref, scat_ref, deg_ref, cf_ref, cb_ref,
                wsf, wsb, wihf, whhf, bihf, bhhf, wihb, whhb, bihb, bhhb,
                *rest):
    scat = scat_ref[...]
    deg = deg_ref[...]
    hf = tf_ref[...]
    hb = tb_ref[...]
    xf = _mmT_hi(scat[0], wsf[...]) + deg[0] * cf_ref[...]
    xb = _mmT_hi(scat[1], wsb[...]) + deg[1] * cb_ref[...]
    hf2 = _gru_half(hf, xf, wihf[...], whhf[...], bihf[...], bhhf[...])
    hb2 = _gru_half(hb, xb, wihb[...], whhb[...], bihb[...], bhhb[...])
    if with_next:
        (wdf, mbf, wdb, mbb, tf_out, tb_out, tgf_out, tgb_out,
         cf_out, cb_out) = rest
        tf_out[...] = hf2
        tb_out[...] = hb2
        tgf_out[...] = _rd(hf2)
        tgb_out[...] = _rd(hb2)
        cf_out[...] = _mmT(hf2, wdf[...]) + mbf[...]
        cb_out[...] = _mmT(hb2, wdb[...]) + mbb[...]
    else:
        (hn_out,) = rest
        hn_out[...] = jnp.concatenate([hf2, hb2], axis=1)


def _tc_layer(tf, tb, scat, deg, c_f, c_b, wsrc, gw, nxt):
    row = pl.BlockSpec((R, NDIM), lambda i: (i, 0))
    row64 = pl.BlockSpec((R, HID), lambda i: (i, 0))
    scat_s = pl.BlockSpec((NC, R, TW), lambda i: (0, i, 0))
    deg_s = pl.BlockSpec((NC, R, 1), lambda i: (0, i, 0))
    wih_s = _full((3, HID, NDIM))
    whh_s = _full((3, HID, HID))
    b_s = _full((3, HID))
    in_specs = [row64, row64, scat_s, deg_s, row, row,
                _full((NDIM, HID)), _full((NDIM, HID)),
                wih_s, whh_s, b_s, b_s, wih_s, whh_s, b_s, b_s]
    args = [tf, tb, scat, deg, c_f, c_b] + list(wsrc) + list(gw)
    o = jax.ShapeDtypeStruct((N, NDIM), F32)
    o64 = jax.ShapeDtypeStruct((N, HID), F32)
    if nxt is not None:
        in_specs += [_full((NDIM, HID)), _full((1, NDIM)),
                     _full((NDIM, HID)), _full((1, NDIM))]
        args += list(nxt)
        out_specs = [row64, row64, row64, row64, row, row]
        out_shape = [o64, o64, o64, o64, o, o]
    else:
        out_specs = [row]
        out_shape = [o]
    return pl.pallas_call(
        functools.partial(_layer_body, nxt is not None),
        grid=(NBLK,),
        in_specs=in_specs,
        out_specs=out_specs,
        out_shape=out_shape,
    )(*args)


def _head1_body(hn_ref, fm_ref, fmb_ref, gm_ref, gmb_ref, out_ref):
    hn = hn_ref[...]
    u = _mmT(hn, fm_ref[0]) + fmb_ref[0]
    g = jax.nn.sigmoid(jnp.sum(hn * gm_ref[0], axis=1, keepdims=True)
                       + gmb_ref[0, 0, 0])
    gated = (u * g).reshape(R // IDX, IDX, GDIM)
    out_ref[0, 0] = jnp.sum(gated, axis=1)


def _tc_head1(hn, fm_s, fmb_s, gm_s, gmb_s):
    return pl.pallas_call(
        _head1_body,
        grid=(2, NBLK),
        in_specs=[pl.BlockSpec((R, NDIM), lambda g, i: (i, 0)),
                  pl.BlockSpec((1, GDIM, NDIM), lambda g, i: (g, 0, 0)),
                  pl.BlockSpec((1, 1, GDIM), lambda g, i: (g, 0, 0)),
                  pl.BlockSpec((1, 1, NDIM), lambda g, i: (g, 0, 0)),
                  pl.BlockSpec((1, 1, 1), lambda g, i: (g, 0, 0))],
        out_specs=pl.BlockSpec((1, 1, R // IDX, GDIM),
                               lambda g, i: (g, i, 0, 0)),
        out_shape=jax.ShapeDtypeStruct((2, NBLK, R // IDX, GDIM), F32),
    )(hn, fm_s, fmb_s, gm_s, gmb_s)


def _head2_body(hg_ref, c_ref, na_ref,
                fw1, fw2, fanb, f2w, f2b, ninits,
                fiw1, fiw2, fiw3, fib, fi2w, fi2b,
                w2, w3, w4, fs1b,
                hv_out, nl_out, t2_out, t34_out):
    hg = hg_ref[0]
    hgi = hg_ref[1]
    c = c_ref[...]
    s = _mmT(hg, fw1[...]) + _mmT(c, fw2[...]) + fanb[...]
    ns = _mmT(jax.nn.relu(s), f2w[...]) + f2b[...]
    m = jnp.max(ns, axis=1, keepdims=True)
    lse = m + jnp.log(jnp.sum(jnp.exp(ns - m), axis=1, keepdims=True))
    logp = ns - lse
    iota = lax.broadcasted_iota(jnp.int32, (B, NUM_ATTS), 1)
    oh = (iota == na_ref[...]).astype(F32)
    nl_out[...] = -jnp.sum(logp * oh, axis=1, keepdims=True)
    e = jnp.dot(oh, ninits[...], preferred_element_type=F32)
    pre = (_mmT(e, fiw1[...]) + _mmT(hgi, fiw2[...]) + _mmT(c, fiw3[...])
           + fib[...])
    hv = _mmT(jax.nn.relu(pre), fi2w[...]) + fi2b[...]
    hv_out[...] = hv
    t2_out[...] = _mmT(hv, w2[...])
    t34_out[...] = _mmT(hg, w3[...]) + _mmT(c, w4[...]) + fs1b[...]


def _tc_head2(hg_s, c, na, weights):
    D2 = NDIM + GDIM
    shapes = [(2, B, GDIM), (B, GDIM), (B, 1),
              (GDIM, GDIM), (GDIM, GDIM), (1, GDIM),
              (NUM_ATTS, GDIM), (1, NUM_ATTS), (NUM_ATTS, NDIM),
              (D2, NDIM), (D2, GDIM), (D2, GDIM), (1, D2),
              (NDIM, D2), (1, NDIM),
              (D2, GDIM), (D2, GDIM), (D2, GDIM), (1, D2)]
    return pl.pallas_call(
        _head2_body,
        in_specs=[_full(s) for s in shapes],
        out_specs=[_full((B, NDIM)), _full((B, 1)),
                   _full((B, D2)), _full((B, D2))],
        out_shape=[jax.ShapeDtypeStruct((B, NDIM), F32),
                   jax.ShapeDtypeStruct((B, 1), F32),
                   jax.ShapeDtypeStruct((B, D2), F32),
                   jax.ShapeDtypeStruct((B, D2), F32)],
    )(hg_s, c, na, *weights)


def _head3_body(hn_ref, w1_ref, t2_ref, t34_ref, f2w_ref, f2b_ref,
                edges_ref, nl_ref, loss_out):
    G = R // IDX
    t1 = _mmT(hn_ref[...], w1_ref[...])
    s2 = (t1.reshape(G, IDX, NDIM + GDIM) + t2_ref[0][:, None, :]
          + t34_ref[...][None, :, :])
    rs = jax.nn.relu(s2)
    es = jnp.sum(rs * f2w_ref[...][None, :, :], axis=2) + f2b_ref[0, 0]
    bce = (jnp.maximum(es, 0.0) - es * edges_ref[0]
           + jnp.log(1.0 + jnp.exp(-jnp.abs(es))))
    el = jnp.mean(bce, axis=1, keepdims=True)
    loss_out[0] = 2.0 * ((1.0 - ALPHA) * nl_ref[0] + ALPHA * el)


def _tc_head3(hn, w1, t2, t34, f2w, f2b, edges, nl):
    D2 = NDIM + GDIM
    G = R // IDX
    return pl.pallas_call(
        _head3_body,
        grid=(NBLK,),
        in_specs=[pl.BlockSpec((R, NDIM), lambda i: (i, 0)),
                  _full((D2, NDIM)),
                  pl.BlockSpec((1, G, D2), lambda i: (i, 0, 0)),
                  _full((B, D2)),
                  _full((1, D2)),
                  _full((1, 1)),
                  pl.BlockSpec((1, G, IDX), lambda i: (i, 0, 0)),
                  pl.BlockSpec((1, G, 1), lambda i: (i, 0, 0))],
        out_specs=pl.BlockSpec((1, G, 1), lambda i: (i, 0, 0)),
        out_shape=jax.ShapeDtypeStruct((NBLK, G, 1), F32),
    )(hn, w1, t2.reshape(NBLK, G, D2), t34, f2w, f2b,
      edges.reshape(NBLK, G, IDX), nl.reshape(NBLK, G, 1))


def _gru_weights(p):
    return (p['Wih'].reshape(3, HID, NDIM), p['Whh'].reshape(3, HID, HID),
            p['bih'].reshape(3, HID), p['bhh'].reshape(3, HID))


def _msg_weights(p):
    return (p['msg_W'][:, :HID], p['msg_W'][:, HID:], p['msg_b'][None, :])


def kernel(h, c, edge_index, node_atts, edges, params):
    h_flat = h.reshape(-1, NDIM)
    ei = edge_index.astype(jnp.int32)
    s0, d0 = ei[0], ei[1]

    padz = jnp.zeros((EP - E,), jnp.int32)
    padd = jnp.full((EP - E,), DUMP, jnp.int32)
    isrc = jnp.stack([jnp.concatenate([s0, padz]),
                      jnp.concatenate([d0, padz])]).reshape(NC, NS, NCH, K)
    idst = jnp.stack([jnp.concatenate([d0, padd]),
                      jnp.concatenate([s0, padd])]).reshape(NC, NS, NCH, K)

    zrow = jnp.zeros((RPT, TW), F32)
    onesw = jnp.ones((K, TW), F32)

    deg = _sc_deg(idst, onesw, zrow)[:, :, :1]

    pf0, pb0 = params['fwd_layers'][0], params['bwd_layers'][0]
    pf1, pb1 = params['fwd_layers'][1], params['bwd_layers'][1]

    wsf0, wdf0, mbf0 = _msg_weights(pf0)
    wsb0, wdb0, mbb0 = _msg_weights(pb0)
    wsf1, wdf1, mbf1 = _msg_weights(pf1)
    wsb1, wdb1, mbb1 = _msg_weights(pb1)
    t_f0, t_b0, tg_f0, tg_b0, c_f0, c_b0 = _tc_pre(h_flat, wdf0, mbf0,
                                                   wdb0, mbb0)

    scat0 = _sc_scatter(tg_f0, tg_b0, isrc, idst, zrow)

    gw0 = _gru_weights(pf0) + _gru_weights(pb0)
    t_f1, t_b1, tg_f1, tg_b1, c_f1, c_b1 = _tc_layer(
        t_f0, t_b0, scat0, deg, c_f0, c_b0,
        (_rd(wsf0), _rd(wsb0)), gw0, (wdf1, mbf1, wdb1, mbb1))

    scat1 = _sc_scatter(tg_f1, tg_b1, isrc, idst, zrow)

    gw1 = _gru_weights(pf1) + _gru_weights(pb1)
    (hn,) = _tc_layer(t_f1, t_b1, scat1, deg, c_f1, c_b1,
                      (_rd(wsf1), _rd(wsb1)), gw1, None)

    pg, pgi = params['graph_emb'], params['graph_emb_init']
    fm_s = jnp.stack([pg['fm_W'], pgi['fm_W']])
    fmb_s = jnp.stack([pg['fm_b'], pgi['fm_b']])[:, None, :]
    gm_s = jnp.stack([pg['gm_W'][0], pgi['gm_W'][0]])[:, None, :]
    gmb_s = jnp.stack([pg['gm_b'], pgi['gm_b']])[:, :, None]
    hg_s = _tc_head1(hn, fm_s, fmb_s, gm_s, gmb_s).reshape(2, B, GDIM)

    D2 = NDIM + GDIM
    f1 = params['fs1_W']
    h2w = (params['fan_W'][:, :GDIM], params['fan_W'][:, GDIM:],
           params['fan_b'][None, :],
           params['fan2_W'], params['fan2_b'][None, :],
           params['node_inits'],
           params['finit_W'][:, :NDIM], params['finit_W'][:, NDIM:NDIM + GDIM],
           params['finit_W'][:, NDIM + GDIM:], params['finit_b'][None, :],
           params['finit2_W'], params['finit2_b'][None, :],
           f1[:, NDIM:NDIM + GDIM], f1[:, NDIM + GDIM:NDIM + 2 * GDIM],
           f1[:, NDIM + 2 * GDIM:], params['fs1_b'][None, :])
    na = node_atts.astype(jnp.int32)[:, None]
    h_v, nl, t2, t34 = _tc_head2(hg_s, c, na, h2w)

    lossb = _tc_head3(hn, f1[:, :NDIM], t2, t34,
                      params['fs2_W'], params['fs2_b'][None, :], edges, nl)

    h_out = jnp.concatenate([hn.reshape(B, IDX, NDIM), h_v[:, None, :]],
                            axis=1)
    return (h_out, lossb.reshape(B))


# speedup vs baseline: 7.8602x; 1.0026x over previous
import functools

import jax
import jax.numpy as jnp
from jax import lax
from jax.experimental import pallas as pl
from jax.experimental.pallas import tpu as pltpu
from jax.experimental.pallas import tpu_sc as plsc

NDIM = 128
GDIM = 128
HID = 64
B = 100
IDX = 100
N = B * IDX
E = 320000
NUM_ATTS = 8
ALPHA = 0.5

NC = 2
NS = 16
K = 128
GC = 8
NCH = 160
NGRP = NCH // GC
ETP = NCH * K
EP = ETP * NS
RPT = 632
NPAD = RPT * NS
DUMP = N
DEGW = 16

F32 = jnp.float32


def _mmT(x, w):
    return lax.dot_general(x, w, (((1,), (1,)), ((), ())),
                           preferred_element_type=F32)


def _mmT_hi(x, w):
    return lax.dot_general(x, w, (((1,), (1,)), ((), ())),
                           precision=lax.Precision.HIGHEST,
                           preferred_element_type=F32)


def _rd(x):
    return x.astype(jnp.bfloat16).astype(F32)


NPAIR = NGRP // 2
TW = HID
DEPTH = 8


def _sc_scatter_body(af_hbm, ab_hbm, isrc_hbm, idst_hbm, zrow_hbm, out_hbm,
                     shared, isrc_a, idst_a, isrc_b, idst_b,
                     rows0, rows1, rows2, rows3, rows4, rows5, rows6, rows7,
                     sem0, sem1, sem2, sem3, sem4, sem5, sem6, sem7):
    cid = lax.axis_index("c")
    sid = lax.axis_index("s")
    pltpu.sync_copy(zrow_hbm, shared.at[pl.ds(sid * RPT, RPT)])
    plsc.subcore_barrier()

    def run(tab):
        rows = (rows0, rows1, rows2, rows3, rows4, rows5, rows6, rows7)
        sems = (sem0, sem1, sem2, sem3, sem4, sem5, sem6, sem7)

        def stage(g, sbuf, dbuf):
            pltpu.sync_copy(isrc_hbm.at[cid, sid, pl.ds(g * GC, GC)], sbuf)
            pltpu.sync_copy(idst_hbm.at[cid, sid, pl.ds(g * GC, GC)], dbuf)

        def issue(sbuf, j, p):
            pltpu.async_copy(tab.at[sbuf.at[j]], rows[p], sems[p])

        def drain(sbuf, j, p):
            pltpu.make_async_copy(tab.at[sbuf.at[j]], rows[p], sems[p]).wait()

        def scat(dbuf, j, p):
            pltpu.sync_copy(rows[p], shared.at[dbuf.at[j]], add=True)

        stage(0, isrc_a, idst_a)
        for p in range(DEPTH):
            issue(isrc_a, p, p)

        def pair(i, carry):
            stage(2 * i + 1, isrc_b, idst_b)
            for j in range(GC):
                p = j % DEPTH
                drain(isrc_a, j, p)
                scat(idst_a, j, p)
                if j + DEPTH < GC:
                    issue(isrc_a, j + DEPTH, p)
                else:
                    issue(isrc_b, j + DEPTH - GC, p)

            @pl.when(i < NPAIR - 1)
            def _():
                stage(2 * i + 2, isrc_a, idst_a)

            for j in range(GC):
                p = j % DEPTH
                drain(isrc_b, j, p)
                scat(idst_b, j, p)
                if j + DEPTH < GC:
                    issue(isrc_b, j + DEPTH, p)
                else:
                    @pl.when(i < NPAIR - 1)
                    def _(j=j, p=p):
                        issue(isrc_a, j + DEPTH - GC, p)
            return carry
        lax.fori_loop(0, NPAIR, pair, 0)

    @pl.when(cid == 0)
    def _():
        run(af_hbm)

    @pl.when(cid == 1)
    def _():
        run(ab_hbm)

    plsc.subcore_barrier()
    pltpu.sync_copy(shared.at[pl.ds(sid * RPT, RPT)],
                    out_hbm.at[cid, pl.ds(sid * RPT, RPT)])


def _sc_scatter(t_f, t_b, isrc, idst, zrow):
    fn = pl.kernel(
        _sc_scatter_body,
        out_type=jax.ShapeDtypeStruct((NC, NPAD, TW), F32),
        compiler_params=pltpu.CompilerParams(use_tc_tiling_on_sc=False),
        mesh=plsc.VectorSubcoreMesh(core_axis_name="c", subcore_axis_name="s"),
        scratch_types=[
            pltpu.VMEM_SHARED((NPAD, TW), F32),
            pltpu.VMEM((GC, K), jnp.int32),
            pltpu.VMEM((GC, K), jnp.int32),
            pltpu.VMEM((GC, K), jnp.int32),
            pltpu.VMEM((GC, K), jnp.int32),
            pltpu.VMEM((K, TW), F32),
            pltpu.VMEM((K, TW), F32),
            pltpu.VMEM((K, TW), F32),
            pltpu.VMEM((K, TW), F32),
            pltpu.VMEM((K, TW), F32),
            pltpu.VMEM((K, TW), F32),
            pltpu.VMEM((K, TW), F32),
            pltpu.VMEM((K, TW), F32),
            pltpu.SemaphoreType.DMA,
            pltpu.SemaphoreType.DMA,
            pltpu.SemaphoreType.DMA,
            pltpu.SemaphoreType.DMA,
            pltpu.SemaphoreType.DMA,
            pltpu.SemaphoreType.DMA,
            pltpu.SemaphoreType.DMA,
            pltpu.SemaphoreType.DMA,
        ],
    )
    return fn(t_f, t_b, isrc, idst, zrow)


def _sc_deg_body(idst_hbm, ones_hbm, zrow_hbm, out_hbm,
                 shared, idst_v, ones_v):
    cid = lax.axis_index("c")
    sid = lax.axis_index("s")
    pltpu.sync_copy(idst_hbm.at[cid, sid], idst_v)
    pltpu.sync_copy(ones_hbm, ones_v)
    pltpu.sync_copy(zrow_hbm, shared.at[pl.ds(sid * RPT, RPT)])
    plsc.subcore_barrier()

    def chunk(j, carry):
        pltpu.sync_copy(ones_v, shared.at[idst_v.at[j]], add=True)
        return carry
    lax.fori_loop(0, NCH, chunk, 0)

    plsc.subcore_barrier()
    pltpu.sync_copy(shared.at[pl.ds(sid * RPT, RPT)],
                    out_hbm.at[cid, pl.ds(sid * RPT, RPT)])


def _sc_deg(idst, ones, zrow):
    fn = pl.kernel(
        _sc_deg_body,
        out_type=jax.ShapeDtypeStruct((NC, NPAD, TW), F32),
        compiler_params=pltpu.CompilerParams(use_tc_tiling_on_sc=False),
        mesh=plsc.VectorSubcoreMesh(core_axis_name="c", subcore_axis_name="s"),
        scratch_types=[
            pltpu.VMEM_SHARED((NPAD, TW), F32),
            pltpu.VMEM((NCH, K), jnp.int32),
            pltpu.VMEM((K, TW), F32),
        ],
    )
    return fn(idst, ones, zrow)


R = 1000
NBLK = N // R


def _full(shape):
    nd = len(shape)
    return pl.BlockSpec(shape, lambda *a, _nd=nd: (0,) * _nd)


def _pre_body(hp_ref, wdf, mbf, wdb, mbb, tf, tb, tgf, tgb, cf, cb):
    hp = hp_ref[...]
    hf = hp[:, :HID]
    hb = hp[:, HID:]
    tf[...] = hf
    tb[...] = hb
    tgf[...] = _rd(hf)
    tgb[...] = _rd(hb)
    cf[...] = _mmT(hf, wdf[...]) + mbf[...]
    cb[...] = _mmT(hb, wdb[...]) + mbb[...]


def _tc_pre(h_flat, wdf, mbf, wdb, mbb):
    row = pl.BlockSpec((R, NDIM), lambda i: (i, 0))
    row64 = pl.BlockSpec((R, HID), lambda i: (i, 0))
    o = jax.ShapeDtypeStruct((N, NDIM), F32)
    o64 = jax.ShapeDtypeStruct((N, HID), F32)
    return pl.pallas_call(
        _pre_body,
        grid=(NBLK,),
        in_specs=[row, _full((NDIM, HID)), _full((1, NDIM)),
                  _full((NDIM, HID)), _full((1, NDIM))],
        out_specs=[row64, row64, row64, row64, row, row],
        out_shape=[o64, o64, o64, o64, o, o],
    )(h_flat, wdf, mbf, wdb, mbb)


def _gru_half(hh, x, wih, whh, bih, bhh):
    r = jax.nn.sigmoid(_mmT(x, wih[0]) + bih[0][None, :]
                       + _mmT(hh, whh[0]) + bhh[0][None, :])
    z = jax.nn.sigmoid(_mmT(x, wih[1]) + bih[1][None, :]
                       + _mmT(hh, whh[1]) + bhh[1][None, :])
    n = jnp.tanh(_mmT(x, wih[2]) + bih[2][None, :]
                 + r * (_mmT(hh, whh[2]) + bhh[2][None, :]))
    return (1.0 - z) * n + z * hh


def _layer_body(with_next, tf_ref, tb_ref, scat_ref, deg_ref, cf_ref, cb_ref,
                wsf, wsb, wihf, whhf, bihf, bhhf, wihb, whhb, bihb, bhhb,
                *rest):
    scat = scat_ref[...]
    deg = deg_ref[...]
    hf = tf_ref[...]
    hb = tb_ref[...]
    xf = _mmT_hi(scat[0], wsf[...]) + deg[0] * cf_ref[...]
    xb = _mmT_hi(scat[1], wsb[...]) + deg[1] * cb_ref[...]
    hf2 = _gru_half(hf, xf, wihf[...], whhf[...], bihf[...], bhhf[...])
    hb2 = _gru_half(hb, xb, wihb[...], whhb[...], bihb[...], bhhb[...])
    if with_next:
        (wdf, mbf, wdb, mbb, tf_out, tb_out, tgf_out, tgb_out,
         cf_out, cb_out) = rest
        tf_out[...] = hf2
        tb_out[...] = hb2
        tgf_out[...] = _rd(hf2)
        tgb_out[...] = _rd(hb2)
        cf_out[...] = _mmT(hf2, wdf[...]) + mbf[...]
        cb_out[...] = _mmT(hb2, wdb[...]) + mbb[...]
    else:
        (hn_out,) = rest
        hn_out[...] = jnp.concatenate([hf2, hb2], axis=1)


def _tc_layer(tf, tb, scat, deg, c_f, c_b, wsrc, gw, nxt):
    row = pl.BlockSpec((R, NDIM), lambda i: (i, 0))
    row64 = pl.BlockSpec((R, HID), lambda i: (i, 0))
    scat_s = pl.BlockSpec((NC, R, TW), lambda i: (0, i, 0))
    deg_s = pl.BlockSpec((NC, R, 1), lambda i: (0, i, 0))
    wih_s = _full((3, HID, NDIM))
    whh_s = _full((3, HID, HID))
    b_s = _full((3, HID))
    in_specs = [row64, row64, scat_s, deg_s, row, row,
                _full((NDIM, HID)), _full((NDIM, HID)),
                wih_s, whh_s, b_s, b_s, wih_s, whh_s, b_s, b_s]
    args = [tf, tb, scat, deg, c_f, c_b] + list(wsrc) + list(gw)
    o = jax.ShapeDtypeStruct((N, NDIM), F32)
    o64 = jax.ShapeDtypeStruct((N, HID), F32)
    if nxt is not None:
        in_specs += [_full((NDIM, HID)), _full((1, NDIM)),
                     _full((NDIM, HID)), _full((1, NDIM))]
        args += list(nxt)
        out_specs = [row64, row64, row64, row64, row, row]
        out_shape = [o64, o64, o64, o64, o, o]
    else:
        out_specs = [row]
        out_shape = [o]
    return pl.pallas_call(
        functools.partial(_layer_body, nxt is not None),
        grid=(NBLK,),
        in_specs=in_specs,
        out_specs=out_specs,
        out_shape=out_shape,
    )(*args)


def _head1_body(hn_ref, fm_ref, fmb_ref, gm_ref, gmb_ref, out_ref):
    hn = hn_ref[...]
    u = _mmT(hn, fm_ref[0]) + fmb_ref[0]
    g = jax.nn.sigmoid(jnp.sum(hn * gm_ref[0], axis=1, keepdims=True)
                       + gmb_ref[0, 0, 0])
    gated = (u * g).reshape(R // IDX, IDX, GDIM)
    out_ref[0, 0] = jnp.sum(gated, axis=1)


def _tc_head1(hn, fm_s, fmb_s, gm_s, gmb_s):
    return pl.pallas_call(
        _head1_body,
        grid=(2, NBLK),
        in_specs=[pl.BlockSpec((R, NDIM), lambda g, i: (i, 0)),
                  pl.BlockSpec((1, GDIM, NDIM), lambda g, i: (g, 0, 0)),
                  pl.BlockSpec((1, 1, GDIM), lambda g, i: (g, 0, 0)),
                  pl.BlockSpec((1, 1, NDIM), lambda g, i: (g, 0, 0)),
                  pl.BlockSpec((1, 1, 1), lambda g, i: (g, 0, 0))],
        out_specs=pl.BlockSpec((1, 1, R // IDX, GDIM),
                               lambda g, i: (g, i, 0, 0)),
        out_shape=jax.ShapeDtypeStruct((2, NBLK, R // IDX, GDIM), F32),
    )(hn, fm_s, fmb_s, gm_s, gmb_s)


def _head2_body(hg_ref, c_ref, na_ref,
                fw1, fw2, fanb, f2w, f2b, ninits,
                fiw1, fiw2, fiw3, fib, fi2w, fi2b,
                w2, w3, w4, fs1b,
                hv_out, nl_out, t2_out, t34_out):
    hg = hg_ref[0]
    hgi = hg_ref[1]
    c = c_ref[...]
    s = _mmT(hg, fw1[...]) + _mmT(c, fw2[...]) + fanb[...]
    ns = _mmT(jax.nn.relu(s), f2w[...]) + f2b[...]
    m = jnp.max(ns, axis=1, keepdims=True)
    lse = m + jnp.log(jnp.sum(jnp.exp(ns - m), axis=1, keepdims=True))
    logp = ns - lse
    iota = lax.broadcasted_iota(jnp.int32, (B, NUM_ATTS), 1)
    oh = (iota == na_ref[...]).astype(F32)
    nl_out[...] = -jnp.sum(logp * oh, axis=1, keepdims=True)
    e = jnp.dot(oh, ninits[...], preferred_element_type=F32)
    pre = (_mmT(e, fiw1[...]) + _mmT(hgi, fiw2[...]) + _mmT(c, fiw3[...])
           + fib[...])
    hv = _mmT(jax.nn.relu(pre), fi2w[...]) + fi2b[...]
    hv_out[...] = hv
    t2_out[...] = _mmT(hv, w2[...])
    t34_out[...] = _mmT(hg, w3[...]) + _mmT(c, w4[...]) + fs1b[...]


def _tc_head2(hg_s, c, na, weights):
    D2 = NDIM + GDIM
    shapes = [(2, B, GDIM), (B, GDIM), (B, 1),
              (GDIM, GDIM), (GDIM, GDIM), (1, GDIM),
              (NUM_ATTS, GDIM), (1, NUM_ATTS), (NUM_ATTS, NDIM),
              (D2, NDIM), (D2, GDIM), (D2, GDIM), (1, D2),
              (NDIM, D2), (1, NDIM),
              (D2, GDIM), (D2, GDIM), (D2, GDIM), (1, D2)]
    return pl.pallas_call(
        _head2_body,
        in_specs=[_full(s) for s in shapes],
        out_specs=[_full((B, NDIM)), _full((B, 1)),
                   _full((B, D2)), _full((B, D2))],
        out_shape=[jax.ShapeDtypeStruct((B, NDIM), F32),
                   jax.ShapeDtypeStruct((B, 1), F32),
                   jax.ShapeDtypeStruct((B, D2), F32),
                   jax.ShapeDtypeStruct((B, D2), F32)],
    )(hg_s, c, na, *weights)


def _head3_body(hn_ref, w1_ref, t2_ref, t34_ref, f2w_ref, f2b_ref,
                edges_ref, nl_ref, loss_out):
    G = R // IDX
    t1 = _mmT(hn_ref[...], w1_ref[...])
    s2 = (t1.reshape(G, IDX, NDIM + GDIM) + t2_ref[0][:, None, :]
          + t34_ref[...][None, :, :])
    rs = jax.nn.relu(s2)
    es = jnp.sum(rs * f2w_ref[...][None, :, :], axis=2) + f2b_ref[0, 0]
    bce = (jnp.maximum(es, 0.0) - es * edges_ref[0]
           + jnp.log(1.0 + jnp.exp(-jnp.abs(es))))
    el = jnp.mean(bce, axis=1, keepdims=True)
    loss_out[0] = 2.0 * ((1.0 - ALPHA) * nl_ref[0] + ALPHA * el)


def _tc_head3(hn, w1, t2, t34, f2w, f2b, edges, nl):
    D2 = NDIM + GDIM
    G = R // IDX
    return pl.pallas_call(
        _head3_body,
        grid=(NBLK,),
        in_specs=[pl.BlockSpec((R, NDIM), lambda i: (i, 0)),
                  _full((D2, NDIM)),
                  pl.BlockSpec((1, G, D2), lambda i: (i, 0, 0)),
                  _full((B, D2)),
                  _full((1, D2)),
                  _full((1, 1)),
                  pl.BlockSpec((1, G, IDX), lambda i: (i, 0, 0)),
                  pl.BlockSpec((1, G, 1), lambda i: (i, 0, 0))],
        out_specs=pl.BlockSpec((1, G, 1), lambda i: (i, 0, 0)),
        out_shape=jax.ShapeDtypeStruct((NBLK, G, 1), F32),
    )(hn, w1, t2.reshape(NBLK, G, D2), t34, f2w, f2b,
      edges.reshape(NBLK, G, IDX), nl.reshape(NBLK, G, 1))


def _gru_weights(p):
    return (p['Wih'].reshape(3, HID, NDIM), p['Whh'].reshape(3, HID, HID),
            p['bih'].reshape(3, HID), p['bhh'].reshape(3, HID))


def _msg_weights(p):
    return (p['msg_W'][:, :HID], p['msg_W'][:, HID:], p['msg_b'][None, :])


def kernel(h, c, edge_index, node_atts, edges, params):
    h_flat = h.reshape(-1, NDIM)
    ei = edge_index.astype(jnp.int32)
    s0, d0 = ei[0], ei[1]

    padz = jnp.zeros((EP - E,), jnp.int32)
    padd = jnp.full((EP - E,), DUMP, jnp.int32)
    isrc = jnp.stack([jnp.concatenate([s0, padz]),
                      jnp.concatenate([d0, padz])]).reshape(NC, NS, NCH, K)
    idst = jnp.stack([jnp.concatenate([d0, padd]),
                      jnp.concatenate([s0, padd])]).reshape(NC, NS, NCH, K)

    zrow = jnp.zeros((RPT, TW), F32)
    onesw = jnp.ones((K, TW), F32)

    deg = _sc_deg(idst, onesw, zrow)[:, :, :1]

    pf0, pb0 = params['fwd_layers'][0], params['bwd_layers'][0]
    pf1, pb1 = params['fwd_layers'][1], params['bwd_layers'][1]

    wsf0, wdf0, mbf0 = _msg_weights(pf0)
    wsb0, wdb0, mbb0 = _msg_weights(pb0)
    wsf1, wdf1, mbf1 = _msg_weights(pf1)
    wsb1, wdb1, mbb1 = _msg_weights(pb1)
    t_f0, t_b0, tg_f0, tg_b0, c_f0, c_b0 = _tc_pre(h_flat, wdf0, mbf0,
                                                   wdb0, mbb0)

    scat0 = _sc_scatter(tg_f0, tg_b0, isrc, idst, zrow)

    gw0 = _gru_weights(pf0) + _gru_weights(pb0)
    t_f1, t_b1, tg_f1, tg_b1, c_f1, c_b1 = _tc_layer(
        t_f0, t_b0, scat0, deg, c_f0, c_b0,
        (_rd(wsf0), _rd(wsb0)), gw0, (wdf1, mbf1, wdb1, mbb1))

    scat1 = _sc_scatter(tg_f1, tg_b1, isrc, idst, zrow)

    gw1 = _gru_weights(pf1) + _gru_weights(pb1)
    (hn,) = _tc_layer(t_f1, t_b1, scat1, deg, c_f1, c_b1,
                      (_rd(wsf1), _rd(wsb1)), gw1, None)

    pg, pgi = params['graph_emb'], params['graph_emb_init']
    fm_s = jnp.stack([pg['fm_W'], pgi['fm_W']])
    fmb_s = jnp.stack([pg['fm_b'], pgi['fm_b']])[:, None, :]
    gm_s = jnp.stack([pg['gm_W'][0], pgi['gm_W'][0]])[:, None, :]
    gmb_s = jnp.stack([pg['gm_b'], pgi['gm_b']])[:, :, None]
    hg_s = _tc_head1(hn, fm_s, fmb_s, gm_s, gmb_s).reshape(2, B, GDIM)

    D2 = NDIM + GDIM
    f1 = params['fs1_W']
    h2w = (params['fan_W'][:, :GDIM], params['fan_W'][:, GDIM:],
           params['fan_b'][None, :],
           params['fan2_W'], params['fan2_b'][None, :],
           params['node_inits'],
           params['finit_W'][:, :NDIM], params['finit_W'][:, NDIM:NDIM + GDIM],
           params['finit_W'][:, NDIM + GDIM:], params['finit_b'][None, :],
           params['finit2_W'], params['finit2_b'][None, :],
           f1[:, NDIM:NDIM + GDIM], f1[:, NDIM + GDIM:NDIM + 2 * GDIM],
           f1[:, NDIM + 2 * GDIM:], params['fs1_b'][None, :])
    na = node_atts.astype(jnp.int32)[:, None]
    h_v, nl, t2, t34 = _tc_head2(hg_s, c, na, h2w)

    lossb = _tc_head3(hn, f1[:, :NDIM], t2, t34,
                      params['fs2_W'], params['fs2_b'][None, :], edges, nl)

    h_out = jnp.concatenate([hn.reshape(B, IDX, NDIM), h_v[:, None, :]],
                            axis=1)
    return (h_out, lossb.reshape(B))
